# baseline (speedup 1.0000x reference)
import jax
import jax.numpy as jnp
from jax.experimental import pallas as pl
from jax.experimental.pallas import tpu as pltpu


def _dma_copy(x_ref, o_ref, sem):
    copy = pltpu.make_async_copy(x_ref, o_ref, sem)
    copy.start()
    copy.wait()


def kernel(X):
    b, n, f = X.shape
    total = b * n * f
    width = 1024
    rows = total // width
    flat = X.reshape(rows, width)
    out = pl.pallas_call(
        _dma_copy,
        in_specs=[pl.BlockSpec(memory_space=pl.ANY)],
        out_specs=pl.BlockSpec(memory_space=pl.ANY),
        out_shape=jax.ShapeDtypeStruct((rows, width), X.dtype),
        scratch_shapes=[pltpu.SemaphoreType.DMA],
    )(flat)
    return out.reshape(b, n, f)
